# Initial kernel scaffold; baseline (speedup 1.0000x reference)
import jax
import jax.numpy as jnp
from jax.experimental import pallas as pl


def kernel(x, edge_index, batch, hom_mask, het_mask, hom_W, hom_b, het_W, het_b, lin1_W, lin1_b, lin2_W, lin2_b, lin3_W, lin3_b, last_epoch):
    raise NotImplementedError("write your pallas kernel here")


# speedup vs baseline: 3.3440x; 3.3440x over previous
import functools

import jax
import jax.numpy as jnp
from jax import lax
from jax.experimental import pallas as pl
from jax.experimental.pallas import tpu as pltpu
from jax.experimental.pallas import tpu_sc as plsc

N = 10000
NPAD = 10240
E = 320000
K = 128
ECH = 2560
EPAD = ECH * K
NSUB = 16
CPT = ECH // NSUB
WCH = CPT // 4
RPT = NPAD // NSUB
D = 128
H = 64
G = 16
CL = 10
BLK = 256
NB = NPAD // BLK

_F32 = jnp.float32
_HI = lax.Precision.HIGHEST


def _mesh():
    return plsc.VectorSubcoreMesh(core_axis_name="c", subcore_axis_name="s",
                                  num_cores=2, num_subcores=NSUB)


def _unpack_dst(ep_v, idx_s, j):
    for r0 in range(0, K, 16):
        e = ep_v[j, pl.ds(r0, 16)]
        idx_s[pl.ds(r0, 16)] = jax.lax.shift_right_logical(e, 16)


def _unpack_src(ep_v, idx_g, j, off):
    for r0 in range(0, K, 16):
        e = ep_v[j, pl.ds(r0, 16)]
        idx_g[pl.ds(r0, 16)] = jnp.bitwise_and(e, 0xFFFF) + off


def _deg_body(ep_h, mask_h, inv_h, ep_v, mask_v, idx_s, zb, invb, dacc):
    c = lax.axis_index("c")
    s = lax.axis_index("s")
    base = s * CPT
    row0 = s * RPT
    pltpu.sync_copy(ep_h.at[pl.ds(base, CPT)], ep_v)
    pltpu.sync_copy(mask_h.at[c, pl.ds(base, CPT)], mask_v)

    def _z(i, carry):
        zb[pl.ds(i * 16, 16)] = jnp.zeros((16,), _F32)
        return carry

    lax.fori_loop(0, RPT // 16, _z, 0)
    pltpu.sync_copy(zb, dacc.at[pl.ds(row0, RPT)])
    plsc.subcore_barrier()

    def _ch(j, carry):
        _unpack_dst(ep_v, idx_s, j)
        pltpu.sync_copy(mask_v.at[j], dacc.at[idx_s], add=True)
        return carry

    lax.fori_loop(0, CPT, _ch, 0)
    plsc.subcore_barrier()
    pltpu.sync_copy(dacc.at[pl.ds(row0, RPT)], invb)

    def _inv(i, carry):
        v = invb[pl.ds(i * 16, 16)]
        invb[pl.ds(i * 16, 16)] = 1.0 / jnp.maximum(v, 1.0)
        return carry

    lax.fori_loop(0, RPT // 16, _inv, 0)
    pltpu.sync_copy(invb, inv_h.at[c, pl.ds(row0, RPT)])


def _make_deg():
    return pl.kernel(
        _deg_body,
        out_type=jax.ShapeDtypeStruct((2, NPAD), _F32),
        mesh=_mesh(),
        compiler_params=pltpu.CompilerParams(needs_layout_passes=False),
        scratch_types=[
            pltpu.VMEM((CPT, K), jnp.int32),
            pltpu.VMEM((CPT, K), _F32),
            pltpu.VMEM((K,), jnp.int32),
            pltpu.VMEM((RPT,), _F32),
            pltpu.VMEM((RPT,), _F32),
            pltpu.VMEM_SHARED((NPAD,), _F32),
        ],
    )


def _splat(ref, idxs):
    vecs = [jnp.zeros((16,), jnp.int32) + i for i in idxs]
    return plsc.load_gather(ref, vecs)


def _layer_body(b_h, ep_h, mask_h, inv_h, s_h, q_h,
                ep_v, mask_v, idx_g, idx_s, g, g1, invd_v,
                acc, sem1):
    c = lax.axis_index("c")
    s = lax.axis_index("s")
    base = s * CPT
    row0 = s * RPT
    iota16 = lax.iota(jnp.int32, 16)

    def _stage_wave(w):
        pltpu.sync_copy(ep_h.at[pl.ds(base + w * WCH, WCH)], ep_v)
        pltpu.sync_copy(mask_h.at[pl.ds(c * ECH + base + w * WCH, WCH)],
                        mask_v)

    pltpu.sync_copy(inv_h.at[c, pl.ds(row0, RPT)], invd_v)

    def _zero_g(i, carry):
        for c0 in range(0, D, 16):
            g[i, pl.ds(c0, 16)] = jnp.zeros((16,), _F32)
        return carry

    def _zero_acc():
        lax.fori_loop(0, K, _zero_g, 0)
        for blk in range(RPT // K):
            pltpu.sync_copy(g, acc.at[pl.ds(row0 + blk * K, K)])

    _zero_acc()
    plsc.subcore_barrier()

    cN = c * NPAD

    def _ident_idx(r0):
        for q in range(0, K, 16):
            idx_g[pl.ds(q, 16)] = iota16 + (r0 + q)

    def _agg_chunk(table_h):
        def _ch(j, carry):
            _unpack_src(ep_v, idx_g, j, cN)
            _unpack_dst(ep_v, idx_s, j)
            cp1 = pltpu.async_copy(table_h.at[idx_g], g, sem1)
            cp1.wait()

            def _sc(r, carry2):
                coef = _splat(mask_v, (j, r))
                for c0 in range(0, D, 16):
                    g[r, pl.ds(c0, 16)] = g[r, pl.ds(c0, 16)] * coef
                return carry2

            lax.fori_loop(0, K, _sc, 0)
            pltpu.sync_copy(g, acc.at[idx_s], add=True)
            return carry
        return _ch

    _h1 = _agg_chunk(b_h)
    for w in range(CPT // WCH):
        _stage_wave(w)
        lax.fori_loop(0, WCH, _h1, 0)
    plsc.subcore_barrier()

    for blk in range(RPT // K):
        r0 = row0 + blk * K
        pltpu.sync_copy(acc.at[pl.ds(r0, K)], g)

        def _qr(r, carry, _blk=blk):
            coef = _splat(invd_v, (_blk * K + r,))
            for c0 in range(0, D, 16):
                g[r, pl.ds(c0, 16)] = g[r, pl.ds(c0, 16)] * coef
            return carry

        lax.fori_loop(0, K, _qr, 0)
        _ident_idx(cN + r0)
        cps = pltpu.async_copy(g, q_h.at[idx_g], sem1)
        cps.wait()

    _zero_acc()
    plsc.subcore_barrier()

    _h2 = _agg_chunk(q_h)
    for w in range(CPT // WCH):
        _stage_wave(w)
        lax.fori_loop(0, WCH, _h2, 0)
    plsc.subcore_barrier()

    for blk in range(RPT // K):
        r0 = row0 + blk * K
        _ident_idx(cN + r0)
        cpq = pltpu.async_copy(q_h.at[idx_g], g1, sem1)
        cpq.wait()
        pltpu.sync_copy(acc.at[pl.ds(r0, K)], g)

        def _fr(r, carry, _blk=blk):
            coef = _splat(invd_v, (_blk * K + r,))
            for c0 in range(0, H, 16):
                g[r, pl.ds(c0, 16)] = (g1[r, pl.ds(c0, 16)] +
                                       g[r, pl.ds(H + c0, 16)] * coef)
            return carry

        lax.fori_loop(0, K, _fr, 0)
        pltpu.sync_copy(g, s_h.at[c, pl.ds(r0, K)])


def _make_layer():
    return pl.kernel(
        _layer_body,
        out_type=(jax.ShapeDtypeStruct((2, NPAD, D), _F32),
                  jax.ShapeDtypeStruct((2 * NPAD, D), _F32)),
        mesh=_mesh(),
        compiler_params=pltpu.CompilerParams(needs_layout_passes=False),
        scratch_types=[
            pltpu.VMEM((WCH, K), jnp.int32),
            pltpu.VMEM((WCH, K), _F32),
            pltpu.VMEM((K,), jnp.int32),
            pltpu.VMEM((K,), jnp.int32),
            pltpu.VMEM((K, D), _F32),
            pltpu.VMEM((K, D), _F32),
            pltpu.VMEM((RPT,), _F32),
            pltpu.VMEM_SHARED((NPAD, D), _F32),
            pltpu.SemaphoreType.DMA,
        ],
    )


def _dot(a, b):
    return jnp.dot(a, b, preferred_element_type=_F32, precision=_HI)


def _prep_body(x_ref, w0_ref, w12_ref, b0_ref, z_ref, b_ref):
    xb = x_ref[...]
    z_ref[...] = (_dot(xb, w0_ref[0]) + b0_ref[0])[None]
    b_ref[...] = _dot(xb, w12_ref[0])[None]


def _readout_accum(g, bt2, gmp, gap, cnt, rb):
    @pl.when(rb == 0)
    def _():
        gmp[...] = jnp.full((G, D), -jnp.inf, _F32)
        gap[...] = jnp.zeros((G, D), _F32)
        cnt[...] = jnp.zeros((G, D), _F32)

    onehot = (bt2 == lax.broadcasted_iota(jnp.int32, (1, G), 1)
              ).astype(_F32)
    gap[...] += lax.dot_general(onehot, g, (((0,), (0,)), ((), ())),
                                preferred_element_type=_F32, precision=_HI)
    cnt[...] += lax.dot_general(onehot, jnp.ones((BLK, D), _F32),
                                (((0,), (0,)), ((), ())),
                                preferred_element_type=_F32, precision=_HI)
    ms = jnp.concatenate(
        [jnp.max(jnp.where(bt2 == gi, g, -jnp.inf), axis=0, keepdims=True)
         for gi in range(G)], axis=0)
    gmp[...] = jnp.maximum(gmp[...], ms)


def _comb_body(do_readout, *refs):
    if do_readout:
        (z_ref, s_ref, w0_ref, w12_ref, b0_ref, batch_ref,
         zn_ref, b_ref, ro_ref, gmp, gap, cnt) = refs
    else:
        (z_ref, s_ref, w0_ref, w12_ref, b0_ref, zn_ref, b_ref) = refs
    c = pl.program_id(0)
    rb = pl.program_id(1)
    sb = s_ref[...]
    zb = z_ref[...]
    g = jnp.maximum(
        jnp.concatenate([zb[0] + sb[0, :, :H], zb[1] + sb[1, :, :H]],
                        axis=1), 0.0)
    zn_ref[...] = (_dot(g, w0_ref[0]) + b0_ref[0])[None]
    b_ref[...] = _dot(g, w12_ref[0])[None]
    if do_readout:
        @pl.when(c == 0)
        def _():
            _readout_accum(g, batch_ref[0], gmp, gap, cnt, rb)

            @pl.when(rb == NB - 1)
            def _():
                ro_ref[...] = jnp.concatenate(
                    [gmp[...], gap[...] / jnp.clip(cnt[...], 1.0, None)],
                    axis=1)


def _final_body(z_ref, s_ref, batch_ref, ro1_ref,
                l1w_ref, l1b_ref, l2w_ref, l2b_ref, l3w_ref, l3b_ref, le_ref,
                out_ref, gmp, gap, cnt):
    rb = pl.program_id(0)
    sb = s_ref[...]
    zb = z_ref[...]
    g = jnp.maximum(
        jnp.concatenate([zb[0] + sb[0, :, :H], zb[1] + sb[1, :, :H]],
                        axis=1), 0.0)
    _readout_accum(g, batch_ref[0], gmp, gap, cnt, rb)

    @pl.when(rb == NB - 1)
    def _():
        ro2 = jnp.concatenate(
            [gmp[...], gap[...] / jnp.clip(cnt[...], 1.0, None)], axis=1)
        r = ro1_ref[...] + ro2
        r = jnp.maximum(_dot(r, l1w_ref[...]) + l1b_ref[...], 0.0)
        r = jnp.maximum(_dot(r, l2w_ref[...]) + l2b_ref[...], 0.0)
        logits = _dot(r, l3w_ref[...]) + l3b_ref[...] + le_ref[0, 0]
        m = jnp.max(logits, axis=1, keepdims=True)
        lse = m + jnp.log(jnp.sum(jnp.exp(logits - m), axis=1, keepdims=True))
        out_ref[...] = logits - lse


def _make_prep():
    return pl.pallas_call(
        _prep_body,
        grid=(2, NB),
        in_specs=[
            pl.BlockSpec((BLK, D), lambda c, rb: (rb, 0)),
            pl.BlockSpec((1, D, H), lambda c, rb: (c, 0, 0)),
            pl.BlockSpec((1, D, D), lambda c, rb: (c, 0, 0)),
            pl.BlockSpec((1, 1, H), lambda c, rb: (c, 0, 0)),
        ],
        out_specs=[
            pl.BlockSpec((1, BLK, H), lambda c, rb: (c, rb, 0)),
            pl.BlockSpec((1, BLK, D), lambda c, rb: (c, rb, 0)),
        ],
        out_shape=[
            jax.ShapeDtypeStruct((2, NPAD, H), _F32),
            jax.ShapeDtypeStruct((2, NPAD, D), _F32),
        ],
    )


def _make_comb(do_readout):
    in_specs = [
        pl.BlockSpec((2, BLK, H), lambda c, rb: (0, rb, 0)),
        pl.BlockSpec((2, BLK, D), lambda c, rb: (0, rb, 0)),
        pl.BlockSpec((1, D, H), lambda c, rb: (c, 0, 0)),
        pl.BlockSpec((1, D, D), lambda c, rb: (c, 0, 0)),
        pl.BlockSpec((1, 1, H), lambda c, rb: (c, 0, 0)),
    ]
    out_specs = [
        pl.BlockSpec((1, BLK, H), lambda c, rb: (c, rb, 0)),
        pl.BlockSpec((1, BLK, D), lambda c, rb: (c, rb, 0)),
    ]
    out_shape = [
        jax.ShapeDtypeStruct((2, NPAD, H), _F32),
        jax.ShapeDtypeStruct((2, NPAD, D), _F32),
    ]
    scratch = []
    if do_readout:
        in_specs.append(pl.BlockSpec((1, BLK, 1), lambda c, rb: (rb, 0, 0)))
        out_specs.append(pl.BlockSpec((G, 2 * D), lambda c, rb: (0, 0)))
        out_shape.append(jax.ShapeDtypeStruct((G, 2 * D), _F32))
        scratch = [pltpu.VMEM((G, D), _F32)] * 3
    return pl.pallas_call(
        functools.partial(_comb_body, do_readout),
        grid=(2, NB),
        in_specs=in_specs,
        out_specs=out_specs,
        out_shape=out_shape,
        scratch_shapes=scratch,
    )


def _make_final():
    return pl.pallas_call(
        _final_body,
        grid=(NB,),
        in_specs=[
            pl.BlockSpec((2, BLK, H), lambda rb: (0, rb, 0)),
            pl.BlockSpec((2, BLK, D), lambda rb: (0, rb, 0)),
            pl.BlockSpec((1, BLK, 1), lambda rb: (rb, 0, 0)),
            pl.BlockSpec((G, 2 * D), lambda rb: (0, 0)),
            pl.BlockSpec((2 * D, D), lambda rb: (0, 0)),
            pl.BlockSpec((1, D), lambda rb: (0, 0)),
            pl.BlockSpec((D, H), lambda rb: (0, 0)),
            pl.BlockSpec((1, H), lambda rb: (0, 0)),
            pl.BlockSpec((H, CL), lambda rb: (0, 0)),
            pl.BlockSpec((1, CL), lambda rb: (0, 0)),
            pl.BlockSpec((1, 1), lambda rb: (0, 0)),
        ],
        out_specs=pl.BlockSpec((G, CL), lambda rb: (0, 0)),
        out_shape=jax.ShapeDtypeStruct((G, CL), _F32),
        scratch_shapes=[pltpu.VMEM((G, D), _F32)] * 3,
    )


def kernel(x, edge_index, batch, hom_mask, het_mask,
           hom_W, hom_b, het_W, het_b,
           lin1_W, lin1_b, lin2_W, lin2_b, lin3_W, lin3_b, last_epoch):
    src = edge_index[0].astype(jnp.int32)
    dst = edge_index[1].astype(jnp.int32)
    pad_idx = (jnp.arange(EPAD - E, dtype=jnp.int32) % N)
    src2 = jnp.concatenate([src, pad_idx])
    dst2 = jnp.concatenate([dst, pad_idx])
    epk = (src2 | (dst2 << 16)).reshape(ECH, K)
    zpad_e = jnp.zeros((EPAD - E,), _F32)
    masks = jnp.stack([
        jnp.concatenate([hom_mask.astype(_F32), zpad_e]),
        jnp.concatenate([het_mask.astype(_F32), zpad_e]),
    ]).reshape(2, ECH, K)
    xp = jnp.zeros((NPAD, D), _F32).at[:N].set(x.astype(_F32))
    batchp = jnp.concatenate(
        [batch.astype(jnp.int32), jnp.full((NPAD - N,), G, jnp.int32)]
    ).reshape(NB, BLK, 1)

    def lw(l):
        w0 = jnp.stack([hom_W[l, 0], het_W[l, 0]]).astype(_F32)
        w12 = jnp.concatenate([
            jnp.stack([hom_W[l, 1], het_W[l, 1]]),
            jnp.stack([hom_W[l, 2], het_W[l, 2]]),
        ], axis=-1).astype(_F32)
        b0 = jnp.stack([hom_b[l], het_b[l]]).astype(_F32).reshape(2, 1, H)
        return w0, w12, b0

    deg_k = _make_deg()
    layer_k = _make_layer()
    prep_k = _make_prep()
    comb_k = _make_comb(False)
    combr_k = _make_comb(True)
    final_k = _make_final()

    invd = deg_k(epk, masks)

    w0, w12, b0 = lw(0)
    z, ba = prep_k(xp, w0, w12, b0)
    masks2 = masks.reshape(2 * ECH, K)
    invd2 = invd
    s_agg, _ = layer_k(ba.reshape(2 * NPAD, D), epk, masks2, invd2)

    w0, w12, b0 = lw(1)
    z, ba = comb_k(z, s_agg, w0, w12, b0)
    s_agg, _ = layer_k(ba.reshape(2 * NPAD, D), epk, masks2, invd2)

    w0, w12, b0 = lw(2)
    z, ba, ro1 = combr_k(z, s_agg, w0, w12, b0, batchp)
    s_agg, _ = layer_k(ba.reshape(2 * NPAD, D), epk, masks2, invd2)

    return final_k(z, s_agg, batchp, ro1,
                   lin1_W.astype(_F32), lin1_b.astype(_F32).reshape(1, D),
                   lin2_W.astype(_F32), lin2_b.astype(_F32).reshape(1, H),
                   lin3_W.astype(_F32), lin3_b.astype(_F32).reshape(1, CL),
                   jnp.asarray(last_epoch, _F32).reshape(1, 1))


# speedup vs baseline: 4.9951x; 1.4938x over previous
import functools

import jax
import jax.numpy as jnp
from jax import lax
from jax.experimental import pallas as pl
from jax.experimental.pallas import tpu as pltpu
from jax.experimental.pallas import tpu_sc as plsc

N = 10000
NPAD = 10240
E = 320000
K = 128
ECH = 2560
EPAD = ECH * K
NSUB = 16
CPT = ECH // NSUB
WCH = CPT // 4
RPT = NPAD // NSUB
D = 128
H = 64
G = 16
CL = 10
BLK = 256
NB = NPAD // BLK

_F32 = jnp.float32
_HI = lax.Precision.HIGHEST


def _mesh():
    return plsc.VectorSubcoreMesh(core_axis_name="c", subcore_axis_name="s",
                                  num_cores=2, num_subcores=NSUB)


def _unpack_dst(ep_v, idx_s, j):
    for r0 in range(0, K, 16):
        e = ep_v[j, pl.ds(r0, 16)]
        idx_s[pl.ds(r0, 16)] = jax.lax.shift_right_logical(e, 16)


def _unpack_src(ep_v, idx_g, j, off):
    for r0 in range(0, K, 16):
        e = ep_v[j, pl.ds(r0, 16)]
        idx_g[pl.ds(r0, 16)] = jnp.bitwise_and(e, 0xFFFF) + off


def _deg_body(ep_h, mask_h, inv_h, ep_v, mask_v, idx_s, zb, invb, dacc):
    c = lax.axis_index("c")
    s = lax.axis_index("s")
    base = s * CPT
    row0 = s * RPT
    pltpu.sync_copy(ep_h.at[pl.ds(base, CPT)], ep_v)
    pltpu.sync_copy(mask_h.at[c, pl.ds(base, CPT)], mask_v)

    def _z(i, carry):
        zb[pl.ds(i * 16, 16)] = jnp.zeros((16,), _F32)
        return carry

    lax.fori_loop(0, RPT // 16, _z, 0)
    pltpu.sync_copy(zb, dacc.at[pl.ds(row0, RPT)])
    plsc.subcore_barrier()

    def _ch(j, carry):
        _unpack_dst(ep_v, idx_s, j)
        pltpu.sync_copy(mask_v.at[j], dacc.at[idx_s], add=True)
        return carry

    lax.fori_loop(0, CPT, _ch, 0)
    plsc.subcore_barrier()
    pltpu.sync_copy(dacc.at[pl.ds(row0, RPT)], invb)

    def _inv(i, carry):
        v = invb[pl.ds(i * 16, 16)]
        invb[pl.ds(i * 16, 16)] = 1.0 / jnp.maximum(v, 1.0)
        return carry

    lax.fori_loop(0, RPT // 16, _inv, 0)
    pltpu.sync_copy(invb, inv_h.at[c, pl.ds(row0, RPT)])


def _make_deg():
    return pl.kernel(
        _deg_body,
        out_type=jax.ShapeDtypeStruct((2, NPAD), _F32),
        mesh=_mesh(),
        compiler_params=pltpu.CompilerParams(needs_layout_passes=False),
        scratch_types=[
            pltpu.VMEM((CPT, K), jnp.int32),
            pltpu.VMEM((CPT, K), _F32),
            pltpu.VMEM((K,), jnp.int32),
            pltpu.VMEM((RPT,), _F32),
            pltpu.VMEM((RPT,), _F32),
            pltpu.VMEM_SHARED((NPAD,), _F32),
        ],
    )


def _splat(ref, idxs):
    vecs = [jnp.zeros((16,), jnp.int32) + i for i in idxs]
    return plsc.load_gather(ref, vecs)


def _layer_body(b_h, ep_h, mask_h, inv_h, s_h, q_h,
                ep_v, mask_v, idx_g, idx_s, idx_g2, idx_s2, g, g1, invd_v,
                acc, sem1, sem2):
    c = lax.axis_index("c")
    s = lax.axis_index("s")
    base = s * CPT
    row0 = s * RPT
    iota16 = lax.iota(jnp.int32, 16)

    def _stage_wave(w):
        pltpu.sync_copy(ep_h.at[pl.ds(base + w * WCH, WCH)], ep_v)
        pltpu.sync_copy(mask_h.at[pl.ds(c * ECH + base + w * WCH, WCH)],
                        mask_v)

    pltpu.sync_copy(inv_h.at[c, pl.ds(row0, RPT)], invd_v)

    def _zero_g(i, carry):
        for c0 in range(0, D, 16):
            g[i, pl.ds(c0, 16)] = jnp.zeros((16,), _F32)
        return carry

    def _zero_acc():
        lax.fori_loop(0, K, _zero_g, 0)
        for blk in range(RPT // K):
            pltpu.sync_copy(g, acc.at[pl.ds(row0 + blk * K, K)])

    _zero_acc()
    plsc.subcore_barrier()

    cN = c * NPAD

    def _ident_idx(r0):
        for q in range(0, K, 16):
            idx_g[pl.ds(q, 16)] = iota16 + (r0 + q)

    def _agg_wave(table_h):
        bufs = ((idx_g, idx_s, g, sem1), (idx_g2, idx_s2, g1, sem2))

        def _start(j, par):
            ig, isc, gb, sem = bufs[par]
            _unpack_src(ep_v, ig, j, cN)
            _unpack_dst(ep_v, isc, j)
            pltpu.async_copy(table_h.at[ig], gb, sem)

        def _finish(j, par):
            ig, isc, gb, sem = bufs[par]
            pltpu.make_async_copy(table_h.at[ig], gb, sem).wait()

            def _sc(r, carry2):
                coef = _splat(mask_v, (j, r))
                for c0 in range(0, D, 16):
                    gb[r, pl.ds(c0, 16)] = gb[r, pl.ds(c0, 16)] * coef
                return carry2

            lax.fori_loop(0, K, _sc, 0)
            pltpu.sync_copy(gb, acc.at[isc], add=True)

        _start(0, 0)

        def _pair(jj, carry):
            j0 = 2 * jj
            _start(j0 + 1, 1)
            _finish(j0, 0)

            @pl.when(jj < WCH // 2 - 1)
            def _():
                _start(j0 + 2, 0)

            _finish(j0 + 1, 1)
            return carry

        lax.fori_loop(0, WCH // 2, _pair, 0)

    for w in range(CPT // WCH):
        _stage_wave(w)
        _agg_wave(b_h)
    plsc.subcore_barrier()

    for blk in range(RPT // K):
        r0 = row0 + blk * K
        pltpu.sync_copy(acc.at[pl.ds(r0, K)], g)

        def _qr(r, carry, _blk=blk):
            coef = _splat(invd_v, (_blk * K + r,))
            for c0 in range(0, D, 16):
                g[r, pl.ds(c0, 16)] = g[r, pl.ds(c0, 16)] * coef
            return carry

        lax.fori_loop(0, K, _qr, 0)
        _ident_idx(cN + r0)
        cps = pltpu.async_copy(g, q_h.at[idx_g], sem1)
        cps.wait()

    _zero_acc()
    plsc.subcore_barrier()

    for w in range(CPT // WCH):
        _stage_wave(w)
        _agg_wave(q_h)
    plsc.subcore_barrier()

    for blk in range(RPT // K):
        r0 = row0 + blk * K
        _ident_idx(cN + r0)
        cpq = pltpu.async_copy(q_h.at[idx_g], g1, sem1)
        cpq.wait()
        pltpu.sync_copy(acc.at[pl.ds(r0, K)], g)

        def _fr(r, carry, _blk=blk):
            coef = _splat(invd_v, (_blk * K + r,))
            for c0 in range(0, H, 16):
                g[r, pl.ds(c0, 16)] = (g1[r, pl.ds(c0, 16)] +
                                       g[r, pl.ds(H + c0, 16)] * coef)
            return carry

        lax.fori_loop(0, K, _fr, 0)
        pltpu.sync_copy(g, s_h.at[c, pl.ds(r0, K)])


def _make_layer():
    return pl.kernel(
        _layer_body,
        out_type=(jax.ShapeDtypeStruct((2, NPAD, D), _F32),
                  jax.ShapeDtypeStruct((2 * NPAD, D), _F32)),
        mesh=_mesh(),
        compiler_params=pltpu.CompilerParams(needs_layout_passes=False),
        scratch_types=[
            pltpu.VMEM((WCH, K), jnp.int32),
            pltpu.VMEM((WCH, K), _F32),
            pltpu.VMEM((K,), jnp.int32),
            pltpu.VMEM((K,), jnp.int32),
            pltpu.VMEM((K,), jnp.int32),
            pltpu.VMEM((K,), jnp.int32),
            pltpu.VMEM((K, D), _F32),
            pltpu.VMEM((K, D), _F32),
            pltpu.VMEM((RPT,), _F32),
            pltpu.VMEM_SHARED((NPAD, D), _F32),
            pltpu.SemaphoreType.DMA,
            pltpu.SemaphoreType.DMA,
        ],
    )


def _dot(a, b):
    return jnp.dot(a, b, preferred_element_type=_F32, precision=_HI)


def _prep_body(x_ref, w0_ref, w12_ref, b0_ref, z_ref, b_ref):
    xb = x_ref[...]
    z_ref[...] = (_dot(xb, w0_ref[0]) + b0_ref[0])[None]
    b_ref[...] = _dot(xb, w12_ref[0])[None]


def _readout_accum(g, bt2, gmp, gap, cnt, rb):
    @pl.when(rb == 0)
    def _():
        gmp[...] = jnp.full((G, D), -jnp.inf, _F32)
        gap[...] = jnp.zeros((G, D), _F32)
        cnt[...] = jnp.zeros((G, D), _F32)

    onehot = (bt2 == lax.broadcasted_iota(jnp.int32, (1, G), 1)
              ).astype(_F32)
    gap[...] += lax.dot_general(onehot, g, (((0,), (0,)), ((), ())),
                                preferred_element_type=_F32, precision=_HI)
    cnt[...] += lax.dot_general(onehot, jnp.ones((BLK, D), _F32),
                                (((0,), (0,)), ((), ())),
                                preferred_element_type=_F32, precision=_HI)
    ms = jnp.concatenate(
        [jnp.max(jnp.where(bt2 == gi, g, -jnp.inf), axis=0, keepdims=True)
         for gi in range(G)], axis=0)
    gmp[...] = jnp.maximum(gmp[...], ms)


def _comb_body(do_readout, *refs):
    if do_readout:
        (z_ref, s_ref, w0_ref, w12_ref, b0_ref, batch_ref,
         zn_ref, b_ref, ro_ref, gmp, gap, cnt) = refs
    else:
        (z_ref, s_ref, w0_ref, w12_ref, b0_ref, zn_ref, b_ref) = refs
    c = pl.program_id(0)
    rb = pl.program_id(1)
    sb = s_ref[...]
    zb = z_ref[...]
    g = jnp.maximum(
        jnp.concatenate([zb[0] + sb[0, :, :H], zb[1] + sb[1, :, :H]],
                        axis=1), 0.0)
    zn_ref[...] = (_dot(g, w0_ref[0]) + b0_ref[0])[None]
    b_ref[...] = _dot(g, w12_ref[0])[None]
    if do_readout:
        @pl.when(c == 0)
        def _():
            _readout_accum(g, batch_ref[0], gmp, gap, cnt, rb)

            @pl.when(rb == NB - 1)
            def _():
                ro_ref[...] = jnp.concatenate(
                    [gmp[...], gap[...] / jnp.clip(cnt[...], 1.0, None)],
                    axis=1)


def _final_body(z_ref, s_ref, batch_ref, ro1_ref,
                l1w_ref, l1b_ref, l2w_ref, l2b_ref, l3w_ref, l3b_ref, le_ref,
                out_ref, gmp, gap, cnt):
    rb = pl.program_id(0)
    sb = s_ref[...]
    zb = z_ref[...]
    g = jnp.maximum(
        jnp.concatenate([zb[0] + sb[0, :, :H], zb[1] + sb[1, :, :H]],
                        axis=1), 0.0)
    _readout_accum(g, batch_ref[0], gmp, gap, cnt, rb)

    @pl.when(rb == NB - 1)
    def _():
        ro2 = jnp.concatenate(
            [gmp[...], gap[...] / jnp.clip(cnt[...], 1.0, None)], axis=1)
        r = ro1_ref[...] + ro2
        r = jnp.maximum(_dot(r, l1w_ref[...]) + l1b_ref[...], 0.0)
        r = jnp.maximum(_dot(r, l2w_ref[...]) + l2b_ref[...], 0.0)
        logits = _dot(r, l3w_ref[...]) + l3b_ref[...] + le_ref[0, 0]
        m = jnp.max(logits, axis=1, keepdims=True)
        lse = m + jnp.log(jnp.sum(jnp.exp(logits - m), axis=1, keepdims=True))
        out_ref[...] = logits - lse


def _make_prep():
    return pl.pallas_call(
        _prep_body,
        grid=(2, NB),
        in_specs=[
            pl.BlockSpec((BLK, D), lambda c, rb: (rb, 0)),
            pl.BlockSpec((1, D, H), lambda c, rb: (c, 0, 0)),
            pl.BlockSpec((1, D, D), lambda c, rb: (c, 0, 0)),
            pl.BlockSpec((1, 1, H), lambda c, rb: (c, 0, 0)),
        ],
        out_specs=[
            pl.BlockSpec((1, BLK, H), lambda c, rb: (c, rb, 0)),
            pl.BlockSpec((1, BLK, D), lambda c, rb: (c, rb, 0)),
        ],
        out_shape=[
            jax.ShapeDtypeStruct((2, NPAD, H), _F32),
            jax.ShapeDtypeStruct((2, NPAD, D), _F32),
        ],
    )


def _make_comb(do_readout):
    in_specs = [
        pl.BlockSpec((2, BLK, H), lambda c, rb: (0, rb, 0)),
        pl.BlockSpec((2, BLK, D), lambda c, rb: (0, rb, 0)),
        pl.BlockSpec((1, D, H), lambda c, rb: (c, 0, 0)),
        pl.BlockSpec((1, D, D), lambda c, rb: (c, 0, 0)),
        pl.BlockSpec((1, 1, H), lambda c, rb: (c, 0, 0)),
    ]
    out_specs = [
        pl.BlockSpec((1, BLK, H), lambda c, rb: (c, rb, 0)),
        pl.BlockSpec((1, BLK, D), lambda c, rb: (c, rb, 0)),
    ]
    out_shape = [
        jax.ShapeDtypeStruct((2, NPAD, H), _F32),
        jax.ShapeDtypeStruct((2, NPAD, D), _F32),
    ]
    scratch = []
    if do_readout:
        in_specs.append(pl.BlockSpec((1, BLK, 1), lambda c, rb: (rb, 0, 0)))
        out_specs.append(pl.BlockSpec((G, 2 * D), lambda c, rb: (0, 0)))
        out_shape.append(jax.ShapeDtypeStruct((G, 2 * D), _F32))
        scratch = [pltpu.VMEM((G, D), _F32)] * 3
    return pl.pallas_call(
        functools.partial(_comb_body, do_readout),
        grid=(2, NB),
        in_specs=in_specs,
        out_specs=out_specs,
        out_shape=out_shape,
        scratch_shapes=scratch,
    )


def _make_final():
    return pl.pallas_call(
        _final_body,
        grid=(NB,),
        in_specs=[
            pl.BlockSpec((2, BLK, H), lambda rb: (0, rb, 0)),
            pl.BlockSpec((2, BLK, D), lambda rb: (0, rb, 0)),
            pl.BlockSpec((1, BLK, 1), lambda rb: (rb, 0, 0)),
            pl.BlockSpec((G, 2 * D), lambda rb: (0, 0)),
            pl.BlockSpec((2 * D, D), lambda rb: (0, 0)),
            pl.BlockSpec((1, D), lambda rb: (0, 0)),
            pl.BlockSpec((D, H), lambda rb: (0, 0)),
            pl.BlockSpec((1, H), lambda rb: (0, 0)),
            pl.BlockSpec((H, CL), lambda rb: (0, 0)),
            pl.BlockSpec((1, CL), lambda rb: (0, 0)),
            pl.BlockSpec((1, 1), lambda rb: (0, 0)),
        ],
        out_specs=pl.BlockSpec((G, CL), lambda rb: (0, 0)),
        out_shape=jax.ShapeDtypeStruct((G, CL), _F32),
        scratch_shapes=[pltpu.VMEM((G, D), _F32)] * 3,
    )


def kernel(x, edge_index, batch, hom_mask, het_mask,
           hom_W, hom_b, het_W, het_b,
           lin1_W, lin1_b, lin2_W, lin2_b, lin3_W, lin3_b, last_epoch):
    src = edge_index[0].astype(jnp.int32)
    dst = edge_index[1].astype(jnp.int32)
    pad_idx = (jnp.arange(EPAD - E, dtype=jnp.int32) % N)
    src2 = jnp.concatenate([src, pad_idx])
    dst2 = jnp.concatenate([dst, pad_idx])
    epk = (src2 | (dst2 << 16)).reshape(ECH, K)
    zpad_e = jnp.zeros((EPAD - E,), _F32)
    masks = jnp.stack([
        jnp.concatenate([hom_mask.astype(_F32), zpad_e]),
        jnp.concatenate([het_mask.astype(_F32), zpad_e]),
    ]).reshape(2, ECH, K)
    xp = jnp.zeros((NPAD, D), _F32).at[:N].set(x.astype(_F32))
    batchp = jnp.concatenate(
        [batch.astype(jnp.int32), jnp.full((NPAD - N,), G, jnp.int32)]
    ).reshape(NB, BLK, 1)

    def lw(l):
        w0 = jnp.stack([hom_W[l, 0], het_W[l, 0]]).astype(_F32)
        w12 = jnp.concatenate([
            jnp.stack([hom_W[l, 1], het_W[l, 1]]),
            jnp.stack([hom_W[l, 2], het_W[l, 2]]),
        ], axis=-1).astype(_F32)
        b0 = jnp.stack([hom_b[l], het_b[l]]).astype(_F32).reshape(2, 1, H)
        return w0, w12, b0

    deg_k = _make_deg()
    layer_k = _make_layer()
    prep_k = _make_prep()
    comb_k = _make_comb(False)
    combr_k = _make_comb(True)
    final_k = _make_final()

    invd = deg_k(epk, masks)

    w0, w12, b0 = lw(0)
    z, ba = prep_k(xp, w0, w12, b0)
    masks2 = masks.reshape(2 * ECH, K)
    invd2 = invd
    s_agg, _ = layer_k(ba.reshape(2 * NPAD, D), epk, masks2, invd2)

    w0, w12, b0 = lw(1)
    z, ba = comb_k(z, s_agg, w0, w12, b0)
    s_agg, _ = layer_k(ba.reshape(2 * NPAD, D), epk, masks2, invd2)

    w0, w12, b0 = lw(2)
    z, ba, ro1 = combr_k(z, s_agg, w0, w12, b0, batchp)
    s_agg, _ = layer_k(ba.reshape(2 * NPAD, D), epk, masks2, invd2)

    return final_k(z, s_agg, batchp, ro1,
                   lin1_W.astype(_F32), lin1_b.astype(_F32).reshape(1, D),
                   lin2_W.astype(_F32), lin2_b.astype(_F32).reshape(1, H),
                   lin3_W.astype(_F32), lin3_b.astype(_F32).reshape(1, CL),
                   jnp.asarray(last_epoch, _F32).reshape(1, 1))


# speedup vs baseline: 5.4341x; 1.0879x over previous
import functools

import jax
import jax.numpy as jnp
from jax import lax
from jax.experimental import pallas as pl
from jax.experimental.pallas import tpu as pltpu
from jax.experimental.pallas import tpu_sc as plsc

N = 10000
NPAD = 10240
E = 320000
K = 128
ECH = 2560
EPAD = ECH * K
NSUB = 16
CPT = ECH // NSUB
WCH = CPT // 4
RPT = NPAD // NSUB
D = 128
H = 64
G = 16
CL = 10
BLK = 256
NB = NPAD // BLK

_F32 = jnp.float32
_HI = lax.Precision.HIGHEST


def _mesh():
    return plsc.VectorSubcoreMesh(core_axis_name="c", subcore_axis_name="s",
                                  num_cores=2, num_subcores=NSUB)


def _unpack_dst(ep_v, idx_s, j):
    for r0 in range(0, K, 16):
        e = ep_v[j, pl.ds(r0, 16)]
        idx_s[pl.ds(r0, 16)] = jax.lax.shift_right_logical(e, 16)


def _unpack_src(ep_v, idx_g, j, off):
    for r0 in range(0, K, 16):
        e = ep_v[j, pl.ds(r0, 16)]
        idx_g[pl.ds(r0, 16)] = jnp.bitwise_and(e, 0xFFFF) + off


def _deg_body(ep_h, mask_h, inv_h, ep_v, mask_v, idx_s, zb, invb, dacc):
    c = lax.axis_index("c")
    s = lax.axis_index("s")
    base = s * CPT
    row0 = s * RPT
    pltpu.sync_copy(ep_h.at[pl.ds(base, CPT)], ep_v)
    pltpu.sync_copy(mask_h.at[c, pl.ds(base, CPT)], mask_v)

    def _z(i, carry):
        zb[pl.ds(i * 16, 16)] = jnp.zeros((16,), _F32)
        return carry

    lax.fori_loop(0, RPT // 16, _z, 0)
    pltpu.sync_copy(zb, dacc.at[pl.ds(row0, RPT)])
    plsc.subcore_barrier()

    def _ch(j, carry):
        _unpack_dst(ep_v, idx_s, j)
        pltpu.sync_copy(mask_v.at[j], dacc.at[idx_s], add=True)
        return carry

    lax.fori_loop(0, CPT, _ch, 0)
    plsc.subcore_barrier()
    pltpu.sync_copy(dacc.at[pl.ds(row0, RPT)], invb)

    def _inv(i, carry):
        v = invb[pl.ds(i * 16, 16)]
        invb[pl.ds(i * 16, 16)] = 1.0 / jnp.maximum(v, 1.0)
        return carry

    lax.fori_loop(0, RPT // 16, _inv, 0)
    pltpu.sync_copy(invb, inv_h.at[c, pl.ds(row0, RPT)])


def _make_deg():
    return pl.kernel(
        _deg_body,
        out_type=jax.ShapeDtypeStruct((2, NPAD), _F32),
        mesh=_mesh(),
        compiler_params=pltpu.CompilerParams(needs_layout_passes=False),
        scratch_types=[
            pltpu.VMEM((CPT, K), jnp.int32),
            pltpu.VMEM((CPT, K), _F32),
            pltpu.VMEM((K,), jnp.int32),
            pltpu.VMEM((RPT,), _F32),
            pltpu.VMEM((RPT,), _F32),
            pltpu.VMEM_SHARED((NPAD,), _F32),
        ],
    )


def _splat(ref, idxs):
    vecs = [jnp.zeros((16,), jnp.int32) + i for i in idxs]
    return plsc.load_gather(ref, vecs)


def _layer_body(b_h, ep_h, mask_h, inv_h, s_h, q_h,
                ep_v, mask_v, idx_g, idx_s, idx_g2, idx_s2, g, g1, invd_v,
                acc, sem1, sem2, sem3, sem4):
    c = lax.axis_index("c")
    s = lax.axis_index("s")
    base = s * CPT
    row0 = s * RPT
    iota16 = lax.iota(jnp.int32, 16)

    def _stage_wave(w):
        pltpu.sync_copy(ep_h.at[pl.ds(base + w * WCH, WCH)], ep_v)
        pltpu.sync_copy(mask_h.at[pl.ds(c * ECH + base + w * WCH, WCH)],
                        mask_v)

    pltpu.sync_copy(inv_h.at[c, pl.ds(row0, RPT)], invd_v)

    def _zero_g(i, carry):
        for c0 in range(0, D, 16):
            g[i, pl.ds(c0, 16)] = jnp.zeros((16,), _F32)
        return carry

    def _zero_acc():
        lax.fori_loop(0, K, _zero_g, 0)
        for blk in range(RPT // K):
            pltpu.sync_copy(g, acc.at[pl.ds(row0 + blk * K, K)])

    _zero_acc()
    plsc.subcore_barrier()

    cN = c * NPAD

    def _ident_idx(r0):
        for q in range(0, K, 16):
            idx_g[pl.ds(q, 16)] = iota16 + (r0 + q)

    def _agg_wave(table_h, c_lo):
        bufs = ((idx_g, idx_s, g, sem1, sem3),
                (idx_g2, idx_s2, g1, sem2, sem4))

        def _start(j, par, wait_scatter):
            ig, isc, gb, semg, sems = bufs[par]
            if wait_scatter:
                pltpu.make_async_copy(gb, acc.at[isc], sems).wait()
            _unpack_src(ep_v, ig, j, cN)
            _unpack_dst(ep_v, isc, j)
            pltpu.async_copy(table_h.at[ig], gb, semg)

        def _finish(j, par):
            ig, isc, gb, semg, sems = bufs[par]
            pltpu.make_async_copy(table_h.at[ig], gb, semg).wait()

            def _sc(r, carry2):
                coef = _splat(mask_v, (j, r))
                for c0 in range(c_lo, D, 16):
                    gb[r, pl.ds(c0, 16)] = gb[r, pl.ds(c0, 16)] * coef
                return carry2

            lax.fori_loop(0, K, _sc, 0)
            pltpu.async_copy(gb, acc.at[isc], sems, add=True)

        _start(0, 0, False)
        _start(1, 1, False)

        def _pair(jj, carry):
            j0 = 2 * jj
            _finish(j0, 0)
            _finish(j0 + 1, 1)
            _start(j0 + 2, 0, True)
            _start(j0 + 3, 1, True)
            return carry

        lax.fori_loop(0, WCH // 2 - 1, _pair, 0)
        _finish(WCH - 2, 0)
        _finish(WCH - 1, 1)
        for par in (0, 1):
            ig, isc, gb, semg, sems = bufs[par]
            pltpu.make_async_copy(gb, acc.at[isc], sems).wait()

    for w in range(CPT // WCH):
        _stage_wave(w)
        _agg_wave(b_h, 0)
    plsc.subcore_barrier()

    for blk in range(RPT // K):
        r0 = row0 + blk * K
        pltpu.sync_copy(acc.at[pl.ds(r0, K)], g)

        def _qr(r, carry, _blk=blk):
            coef = _splat(invd_v, (_blk * K + r,))
            for c0 in range(0, D, 16):
                g[r, pl.ds(c0, 16)] = g[r, pl.ds(c0, 16)] * coef
            return carry

        lax.fori_loop(0, K, _qr, 0)
        _ident_idx(cN + r0)
        cps = pltpu.async_copy(g, q_h.at[idx_g], sem1)
        cps.wait()

    _zero_acc()
    plsc.subcore_barrier()

    for w in range(CPT // WCH):
        _stage_wave(w)
        _agg_wave(q_h, H)
    plsc.subcore_barrier()

    for blk in range(RPT // K):
        r0 = row0 + blk * K
        _ident_idx(cN + r0)
        cpq = pltpu.async_copy(q_h.at[idx_g], g1, sem1)
        cpq.wait()
        pltpu.sync_copy(acc.at[pl.ds(r0, K)], g)

        def _fr(r, carry, _blk=blk):
            coef = _splat(invd_v, (_blk * K + r,))
            for c0 in range(0, H, 16):
                g[r, pl.ds(c0, 16)] = (g1[r, pl.ds(c0, 16)] +
                                       g[r, pl.ds(H + c0, 16)] * coef)
            return carry

        lax.fori_loop(0, K, _fr, 0)
        pltpu.sync_copy(g, s_h.at[c, pl.ds(r0, K)])


def _make_layer():
    return pl.kernel(
        _layer_body,
        out_type=(jax.ShapeDtypeStruct((2, NPAD, D), _F32),
                  jax.ShapeDtypeStruct((2 * NPAD, D), _F32)),
        mesh=_mesh(),
        compiler_params=pltpu.CompilerParams(needs_layout_passes=False),
        scratch_types=[
            pltpu.VMEM((WCH, K), jnp.int32),
            pltpu.VMEM((WCH, K), _F32),
            pltpu.VMEM((K,), jnp.int32),
            pltpu.VMEM((K,), jnp.int32),
            pltpu.VMEM((K,), jnp.int32),
            pltpu.VMEM((K,), jnp.int32),
            pltpu.VMEM((K, D), _F32),
            pltpu.VMEM((K, D), _F32),
            pltpu.VMEM((RPT,), _F32),
            pltpu.VMEM_SHARED((NPAD, D), _F32),
            pltpu.SemaphoreType.DMA,
            pltpu.SemaphoreType.DMA,
            pltpu.SemaphoreType.DMA,
            pltpu.SemaphoreType.DMA,
        ],
    )


def _dot(a, b):
    return jnp.dot(a, b, preferred_element_type=_F32, precision=_HI)


def _prep_body(x_ref, w0_ref, w12_ref, b0_ref, z_ref, b_ref):
    xb = x_ref[...]
    z_ref[...] = (_dot(xb, w0_ref[0]) + b0_ref[0])[None]
    b_ref[...] = _dot(xb, w12_ref[0])[None]


def _readout_accum(g, bt2, gmp, gap, cnt, rb):
    @pl.when(rb == 0)
    def _():
        gmp[...] = jnp.full((G, D), -jnp.inf, _F32)
        gap[...] = jnp.zeros((G, D), _F32)
        cnt[...] = jnp.zeros((G, D), _F32)

    onehot = (bt2 == lax.broadcasted_iota(jnp.int32, (1, G), 1)
              ).astype(_F32)
    gap[...] += lax.dot_general(onehot, g, (((0,), (0,)), ((), ())),
                                preferred_element_type=_F32, precision=_HI)
    cnt[...] += lax.dot_general(onehot, jnp.ones((BLK, D), _F32),
                                (((0,), (0,)), ((), ())),
                                preferred_element_type=_F32, precision=_HI)
    ms = jnp.concatenate(
        [jnp.max(jnp.where(bt2 == gi, g, -jnp.inf), axis=0, keepdims=True)
         for gi in range(G)], axis=0)
    gmp[...] = jnp.maximum(gmp[...], ms)


def _comb_body(do_readout, *refs):
    if do_readout:
        (z_ref, s_ref, w0_ref, w12_ref, b0_ref, batch_ref,
         zn_ref, b_ref, ro_ref, gmp, gap, cnt) = refs
    else:
        (z_ref, s_ref, w0_ref, w12_ref, b0_ref, zn_ref, b_ref) = refs
    c = pl.program_id(0)
    rb = pl.program_id(1)
    sb = s_ref[...]
    zb = z_ref[...]
    g = jnp.maximum(
        jnp.concatenate([zb[0] + sb[0, :, :H], zb[1] + sb[1, :, :H]],
                        axis=1), 0.0)
    zn_ref[...] = (_dot(g, w0_ref[0]) + b0_ref[0])[None]
    b_ref[...] = _dot(g, w12_ref[0])[None]
    if do_readout:
        @pl.when(c == 0)
        def _():
            _readout_accum(g, batch_ref[0], gmp, gap, cnt, rb)

            @pl.when(rb == NB - 1)
            def _():
                ro_ref[...] = jnp.concatenate(
                    [gmp[...], gap[...] / jnp.clip(cnt[...], 1.0, None)],
                    axis=1)


def _final_body(z_ref, s_ref, batch_ref, ro1_ref,
                l1w_ref, l1b_ref, l2w_ref, l2b_ref, l3w_ref, l3b_ref, le_ref,
                out_ref, gmp, gap, cnt):
    rb = pl.program_id(0)
    sb = s_ref[...]
    zb = z_ref[...]
    g = jnp.maximum(
        jnp.concatenate([zb[0] + sb[0, :, :H], zb[1] + sb[1, :, :H]],
                        axis=1), 0.0)
    _readout_accum(g, batch_ref[0], gmp, gap, cnt, rb)

    @pl.when(rb == NB - 1)
    def _():
        ro2 = jnp.concatenate(
            [gmp[...], gap[...] / jnp.clip(cnt[...], 1.0, None)], axis=1)
        r = ro1_ref[...] + ro2
        r = jnp.maximum(_dot(r, l1w_ref[...]) + l1b_ref[...], 0.0)
        r = jnp.maximum(_dot(r, l2w_ref[...]) + l2b_ref[...], 0.0)
        logits = _dot(r, l3w_ref[...]) + l3b_ref[...] + le_ref[0, 0]
        m = jnp.max(logits, axis=1, keepdims=True)
        lse = m + jnp.log(jnp.sum(jnp.exp(logits - m), axis=1, keepdims=True))
        out_ref[...] = logits - lse


def _make_prep():
    return pl.pallas_call(
        _prep_body,
        grid=(2, NB),
        in_specs=[
            pl.BlockSpec((BLK, D), lambda c, rb: (rb, 0)),
            pl.BlockSpec((1, D, H), lambda c, rb: (c, 0, 0)),
            pl.BlockSpec((1, D, D), lambda c, rb: (c, 0, 0)),
            pl.BlockSpec((1, 1, H), lambda c, rb: (c, 0, 0)),
        ],
        out_specs=[
            pl.BlockSpec((1, BLK, H), lambda c, rb: (c, rb, 0)),
            pl.BlockSpec((1, BLK, D), lambda c, rb: (c, rb, 0)),
        ],
        out_shape=[
            jax.ShapeDtypeStruct((2, NPAD, H), _F32),
            jax.ShapeDtypeStruct((2, NPAD, D), _F32),
        ],
    )


def _make_comb(do_readout):
    in_specs = [
        pl.BlockSpec((2, BLK, H), lambda c, rb: (0, rb, 0)),
        pl.BlockSpec((2, BLK, D), lambda c, rb: (0, rb, 0)),
        pl.BlockSpec((1, D, H), lambda c, rb: (c, 0, 0)),
        pl.BlockSpec((1, D, D), lambda c, rb: (c, 0, 0)),
        pl.BlockSpec((1, 1, H), lambda c, rb: (c, 0, 0)),
    ]
    out_specs = [
        pl.BlockSpec((1, BLK, H), lambda c, rb: (c, rb, 0)),
        pl.BlockSpec((1, BLK, D), lambda c, rb: (c, rb, 0)),
    ]
    out_shape = [
        jax.ShapeDtypeStruct((2, NPAD, H), _F32),
        jax.ShapeDtypeStruct((2, NPAD, D), _F32),
    ]
    scratch = []
    if do_readout:
        in_specs.append(pl.BlockSpec((1, BLK, 1), lambda c, rb: (rb, 0, 0)))
        out_specs.append(pl.BlockSpec((G, 2 * D), lambda c, rb: (0, 0)))
        out_shape.append(jax.ShapeDtypeStruct((G, 2 * D), _F32))
        scratch = [pltpu.VMEM((G, D), _F32)] * 3
    return pl.pallas_call(
        functools.partial(_comb_body, do_readout),
        grid=(2, NB),
        in_specs=in_specs,
        out_specs=out_specs,
        out_shape=out_shape,
        scratch_shapes=scratch,
    )


def _make_final():
    return pl.pallas_call(
        _final_body,
        grid=(NB,),
        in_specs=[
            pl.BlockSpec((2, BLK, H), lambda rb: (0, rb, 0)),
            pl.BlockSpec((2, BLK, D), lambda rb: (0, rb, 0)),
            pl.BlockSpec((1, BLK, 1), lambda rb: (rb, 0, 0)),
            pl.BlockSpec((G, 2 * D), lambda rb: (0, 0)),
            pl.BlockSpec((2 * D, D), lambda rb: (0, 0)),
            pl.BlockSpec((1, D), lambda rb: (0, 0)),
            pl.BlockSpec((D, H), lambda rb: (0, 0)),
            pl.BlockSpec((1, H), lambda rb: (0, 0)),
            pl.BlockSpec((H, CL), lambda rb: (0, 0)),
            pl.BlockSpec((1, CL), lambda rb: (0, 0)),
            pl.BlockSpec((1, 1), lambda rb: (0, 0)),
        ],
        out_specs=pl.BlockSpec((G, CL), lambda rb: (0, 0)),
        out_shape=jax.ShapeDtypeStruct((G, CL), _F32),
        scratch_shapes=[pltpu.VMEM((G, D), _F32)] * 3,
    )


def kernel(x, edge_index, batch, hom_mask, het_mask,
           hom_W, hom_b, het_W, het_b,
           lin1_W, lin1_b, lin2_W, lin2_b, lin3_W, lin3_b, last_epoch):
    src = edge_index[0].astype(jnp.int32)
    dst = edge_index[1].astype(jnp.int32)
    pad_idx = (jnp.arange(EPAD - E, dtype=jnp.int32) % N)
    src2 = jnp.concatenate([src, pad_idx])
    dst2 = jnp.concatenate([dst, pad_idx])
    epk = (src2 | (dst2 << 16)).reshape(ECH, K)
    zpad_e = jnp.zeros((EPAD - E,), _F32)
    masks = jnp.stack([
        jnp.concatenate([hom_mask.astype(_F32), zpad_e]),
        jnp.concatenate([het_mask.astype(_F32), zpad_e]),
    ]).reshape(2, ECH, K)
    xp = jnp.zeros((NPAD, D), _F32).at[:N].set(x.astype(_F32))
    batchp = jnp.concatenate(
        [batch.astype(jnp.int32), jnp.full((NPAD - N,), G, jnp.int32)]
    ).reshape(NB, BLK, 1)

    def lw(l):
        w0 = jnp.stack([hom_W[l, 0], het_W[l, 0]]).astype(_F32)
        w12 = jnp.concatenate([
            jnp.stack([hom_W[l, 1], het_W[l, 1]]),
            jnp.stack([hom_W[l, 2], het_W[l, 2]]),
        ], axis=-1).astype(_F32)
        b0 = jnp.stack([hom_b[l], het_b[l]]).astype(_F32).reshape(2, 1, H)
        return w0, w12, b0

    deg_k = _make_deg()
    layer_k = _make_layer()
    prep_k = _make_prep()
    comb_k = _make_comb(False)
    combr_k = _make_comb(True)
    final_k = _make_final()

    invd = deg_k(epk, masks)

    w0, w12, b0 = lw(0)
    z, ba = prep_k(xp, w0, w12, b0)
    masks2 = masks.reshape(2 * ECH, K)
    invd2 = invd
    s_agg, _ = layer_k(ba.reshape(2 * NPAD, D), epk, masks2, invd2)

    w0, w12, b0 = lw(1)
    z, ba = comb_k(z, s_agg, w0, w12, b0)
    s_agg, _ = layer_k(ba.reshape(2 * NPAD, D), epk, masks2, invd2)

    w0, w12, b0 = lw(2)
    z, ba, ro1 = combr_k(z, s_agg, w0, w12, b0, batchp)
    s_agg, _ = layer_k(ba.reshape(2 * NPAD, D), epk, masks2, invd2)

    return final_k(z, s_agg, batchp, ro1,
                   lin1_W.astype(_F32), lin1_b.astype(_F32).reshape(1, D),
                   lin2_W.astype(_F32), lin2_b.astype(_F32).reshape(1, H),
                   lin3_W.astype(_F32), lin3_b.astype(_F32).reshape(1, CL),
                   jnp.asarray(last_epoch, _F32).reshape(1, 1))


# speedup vs baseline: 5.5387x; 1.0192x over previous
import functools

import jax
import jax.numpy as jnp
from jax import lax
from jax.experimental import pallas as pl
from jax.experimental.pallas import tpu as pltpu
from jax.experimental.pallas import tpu_sc as plsc

N = 10000
NPAD = 10240
E = 320000
K = 128
ECH = 2560
EPAD = ECH * K
NSUB = 16
CPT = ECH // NSUB
WCH = CPT // 4
RPT = NPAD // NSUB
D = 128
H = 64
G = 16
CL = 10
BLK = 256
NB = NPAD // BLK

_F32 = jnp.float32
_HI = lax.Precision.HIGHEST


def _mesh():
    return plsc.VectorSubcoreMesh(core_axis_name="c", subcore_axis_name="s",
                                  num_cores=2, num_subcores=NSUB)


def _unpack_dst(ep_v, idx_s, j):
    for r0 in range(0, K, 16):
        e = ep_v[j, pl.ds(r0, 16)]
        idx_s[pl.ds(r0, 16)] = jax.lax.shift_right_logical(e, 16)


def _unpack_src(ep_v, idx_g, j, off):
    for r0 in range(0, K, 16):
        e = ep_v[j, pl.ds(r0, 16)]
        idx_g[pl.ds(r0, 16)] = jnp.bitwise_and(e, 0xFFFF) + off


def _deg_body(ep_h, mask_h, inv_h, ep_v, mask_v, idx_s, zb, invb, dacc):
    c = lax.axis_index("c")
    s = lax.axis_index("s")
    base = s * CPT
    row0 = s * RPT
    pltpu.sync_copy(ep_h.at[pl.ds(base, CPT)], ep_v)
    pltpu.sync_copy(mask_h.at[c, pl.ds(base, CPT)], mask_v)

    def _z(i, carry):
        zb[pl.ds(i * 16, 16)] = jnp.zeros((16,), _F32)
        return carry

    lax.fori_loop(0, RPT // 16, _z, 0)
    pltpu.sync_copy(zb, dacc.at[pl.ds(row0, RPT)])
    plsc.subcore_barrier()

    def _ch(j, carry):
        _unpack_dst(ep_v, idx_s, j)
        pltpu.sync_copy(mask_v.at[j], dacc.at[idx_s], add=True)
        return carry

    lax.fori_loop(0, CPT, _ch, 0)
    plsc.subcore_barrier()
    pltpu.sync_copy(dacc.at[pl.ds(row0, RPT)], invb)

    def _inv(i, carry):
        v = invb[pl.ds(i * 16, 16)]
        invb[pl.ds(i * 16, 16)] = 1.0 / jnp.maximum(v, 1.0)
        return carry

    lax.fori_loop(0, RPT // 16, _inv, 0)
    pltpu.sync_copy(invb, inv_h.at[c, pl.ds(row0, RPT)])


def _make_deg():
    return pl.kernel(
        _deg_body,
        out_type=jax.ShapeDtypeStruct((2, NPAD), _F32),
        mesh=_mesh(),
        compiler_params=pltpu.CompilerParams(needs_layout_passes=False),
        scratch_types=[
            pltpu.VMEM((CPT, K), jnp.int32),
            pltpu.VMEM((CPT, K), _F32),
            pltpu.VMEM((K,), jnp.int32),
            pltpu.VMEM((RPT,), _F32),
            pltpu.VMEM((RPT,), _F32),
            pltpu.VMEM_SHARED((NPAD,), _F32),
        ],
    )


def _splat(ref, idxs):
    vecs = [jnp.zeros((16,), jnp.int32) + i for i in idxs]
    return plsc.load_gather(ref, vecs)


def _layer_body(b_h, ep_h, mask_h, inv_h, s_h, q_h,
                ep_v, mask_v, idx_g, idx_s, idx_g2, idx_s2, g, g1, invd_v,
                acc, sem1, sem2, sem3, sem4):
    c = lax.axis_index("c")
    s = lax.axis_index("s")
    base = s * CPT
    row0 = s * RPT
    iota16 = lax.iota(jnp.int32, 16)

    def _stage_wave(w):
        pltpu.sync_copy(ep_h.at[pl.ds(base + w * WCH, WCH)], ep_v)
        pltpu.sync_copy(mask_h.at[pl.ds(c * ECH + base + w * WCH, WCH)],
                        mask_v)

    pltpu.sync_copy(inv_h.at[c, pl.ds(row0, RPT)], invd_v)

    def _zero_g(i, carry):
        for c0 in range(0, D, 16):
            g[i, pl.ds(c0, 16)] = jnp.zeros((16,), _F32)
        return carry

    def _zero_acc():
        lax.fori_loop(0, K, _zero_g, 0)
        for blk in range(RPT // K):
            pltpu.sync_copy(g, acc.at[pl.ds(row0 + blk * K, K)])

    _zero_acc()
    plsc.subcore_barrier()

    cN = c * NPAD

    def _ident_idx(r0):
        for q in range(0, K, 16):
            idx_g[pl.ds(q, 16)] = iota16 + (r0 + q)

    def _agg_wave(table_h, c_lo):
        bufs = ((idx_g, idx_s, g, sem1, sem3),
                (idx_g2, idx_s2, g1, sem2, sem4))

        def _start(j, par, wait_scatter):
            ig, isc, gb, semg, sems = bufs[par]
            if wait_scatter:
                pltpu.make_async_copy(gb, acc.at[isc], sems).wait()
            _unpack_src(ep_v, ig, j, cN)
            _unpack_dst(ep_v, isc, j)
            pltpu.async_copy(table_h.at[ig], gb, semg)

        def _finish(j, par):
            ig, isc, gb, semg, sems = bufs[par]
            pltpu.make_async_copy(table_h.at[ig], gb, semg).wait()

            def _sc(rr, carry2):
                for k in range(4):
                    r = rr * 4 + k
                    coef = _splat(mask_v, (j, r))
                    for c0 in range(c_lo, D, 16):
                        gb[r, pl.ds(c0, 16)] = gb[r, pl.ds(c0, 16)] * coef
                return carry2

            lax.fori_loop(0, K // 4, _sc, 0)
            pltpu.async_copy(gb, acc.at[isc], sems, add=True)

        _start(0, 0, False)
        _start(1, 1, False)

        def _pair(jj, carry):
            j0 = 2 * jj
            _finish(j0, 0)
            _finish(j0 + 1, 1)
            _start(j0 + 2, 0, True)
            _start(j0 + 3, 1, True)
            return carry

        lax.fori_loop(0, WCH // 2 - 1, _pair, 0)
        _finish(WCH - 2, 0)
        _finish(WCH - 1, 1)
        for par in (0, 1):
            ig, isc, gb, semg, sems = bufs[par]
            pltpu.make_async_copy(gb, acc.at[isc], sems).wait()

    for w in range(CPT // WCH):
        _stage_wave(w)
        _agg_wave(b_h, 0)
    plsc.subcore_barrier()

    for blk in range(RPT // K):
        r0 = row0 + blk * K
        pltpu.sync_copy(acc.at[pl.ds(r0, K)], g)

        def _qr(rr, carry, _blk=blk):
            for k in range(4):
                r = rr * 4 + k
                coef = _splat(invd_v, (_blk * K + r,))
                for c0 in range(0, D, 16):
                    g[r, pl.ds(c0, 16)] = g[r, pl.ds(c0, 16)] * coef
            return carry

        lax.fori_loop(0, K // 4, _qr, 0)
        _ident_idx(cN + r0)
        cps = pltpu.async_copy(g, q_h.at[idx_g], sem1)
        cps.wait()

    _zero_acc()
    plsc.subcore_barrier()

    for w in range(CPT // WCH):
        _stage_wave(w)
        _agg_wave(q_h, H)
    plsc.subcore_barrier()

    for blk in range(RPT // K):
        r0 = row0 + blk * K
        _ident_idx(cN + r0)
        cpq = pltpu.async_copy(q_h.at[idx_g], g1, sem1)
        cpq.wait()
        pltpu.sync_copy(acc.at[pl.ds(r0, K)], g)

        def _fr(rr, carry, _blk=blk):
            for k in range(4):
                r = rr * 4 + k
                coef = _splat(invd_v, (_blk * K + r,))
                for c0 in range(0, H, 16):
                    g[r, pl.ds(c0, 16)] = (g1[r, pl.ds(c0, 16)] +
                                           g[r, pl.ds(H + c0, 16)] * coef)
            return carry

        lax.fori_loop(0, K // 4, _fr, 0)
        pltpu.sync_copy(g, s_h.at[c, pl.ds(r0, K)])


def _make_layer():
    return pl.kernel(
        _layer_body,
        out_type=(jax.ShapeDtypeStruct((2, NPAD, D), _F32),
                  jax.ShapeDtypeStruct((2 * NPAD, D), _F32)),
        mesh=_mesh(),
        compiler_params=pltpu.CompilerParams(needs_layout_passes=False),
        scratch_types=[
            pltpu.VMEM((WCH, K), jnp.int32),
            pltpu.VMEM((WCH, K), _F32),
            pltpu.VMEM((K,), jnp.int32),
            pltpu.VMEM((K,), jnp.int32),
            pltpu.VMEM((K,), jnp.int32),
            pltpu.VMEM((K,), jnp.int32),
            pltpu.VMEM((K, D), _F32),
            pltpu.VMEM((K, D), _F32),
            pltpu.VMEM((RPT,), _F32),
            pltpu.VMEM_SHARED((NPAD, D), _F32),
            pltpu.SemaphoreType.DMA,
            pltpu.SemaphoreType.DMA,
            pltpu.SemaphoreType.DMA,
            pltpu.SemaphoreType.DMA,
        ],
    )


def _dot(a, b):
    return jnp.dot(a, b, preferred_element_type=_F32, precision=_HI)


def _prep_body(x_ref, w0_ref, w12_ref, b0_ref, z_ref, b_ref):
    xb = x_ref[...]
    z_ref[...] = (_dot(xb, w0_ref[0]) + b0_ref[0])[None]
    b_ref[...] = _dot(xb, w12_ref[0])[None]


def _readout_accum(g, bt2, gmp, gap, cnt, rb):
    @pl.when(rb == 0)
    def _():
        gmp[...] = jnp.full((G, D), -jnp.inf, _F32)
        gap[...] = jnp.zeros((G, D), _F32)
        cnt[...] = jnp.zeros((G, D), _F32)

    onehot = (bt2 == lax.broadcasted_iota(jnp.int32, (1, G), 1)
              ).astype(_F32)
    gap[...] += lax.dot_general(onehot, g, (((0,), (0,)), ((), ())),
                                preferred_element_type=_F32, precision=_HI)
    cnt[...] += lax.dot_general(onehot, jnp.ones((BLK, D), _F32),
                                (((0,), (0,)), ((), ())),
                                preferred_element_type=_F32, precision=_HI)
    ms = jnp.concatenate(
        [jnp.max(jnp.where(bt2 == gi, g, -jnp.inf), axis=0, keepdims=True)
         for gi in range(G)], axis=0)
    gmp[...] = jnp.maximum(gmp[...], ms)


def _comb_body(do_readout, *refs):
    if do_readout:
        (z_ref, s_ref, w0_ref, w12_ref, b0_ref, batch_ref,
         zn_ref, b_ref, ro_ref, gmp, gap, cnt) = refs
    else:
        (z_ref, s_ref, w0_ref, w12_ref, b0_ref, zn_ref, b_ref) = refs
    c = pl.program_id(0)
    rb = pl.program_id(1)
    sb = s_ref[...]
    zb = z_ref[...]
    g = jnp.maximum(
        jnp.concatenate([zb[0] + sb[0, :, :H], zb[1] + sb[1, :, :H]],
                        axis=1), 0.0)
    zn_ref[...] = (_dot(g, w0_ref[0]) + b0_ref[0])[None]
    b_ref[...] = _dot(g, w12_ref[0])[None]
    if do_readout:
        @pl.when(c == 0)
        def _():
            _readout_accum(g, batch_ref[0], gmp, gap, cnt, rb)

            @pl.when(rb == NB - 1)
            def _():
                ro_ref[...] = jnp.concatenate(
                    [gmp[...], gap[...] / jnp.clip(cnt[...], 1.0, None)],
                    axis=1)


def _final_body(z_ref, s_ref, batch_ref, ro1_ref,
                l1w_ref, l1b_ref, l2w_ref, l2b_ref, l3w_ref, l3b_ref, le_ref,
                out_ref, gmp, gap, cnt):
    rb = pl.program_id(0)
    sb = s_ref[...]
    zb = z_ref[...]
    g = jnp.maximum(
        jnp.concatenate([zb[0] + sb[0, :, :H], zb[1] + sb[1, :, :H]],
                        axis=1), 0.0)
    _readout_accum(g, batch_ref[0], gmp, gap, cnt, rb)

    @pl.when(rb == NB - 1)
    def _():
        ro2 = jnp.concatenate(
            [gmp[...], gap[...] / jnp.clip(cnt[...], 1.0, None)], axis=1)
        r = ro1_ref[...] + ro2
        r = jnp.maximum(_dot(r, l1w_ref[...]) + l1b_ref[...], 0.0)
        r = jnp.maximum(_dot(r, l2w_ref[...]) + l2b_ref[...], 0.0)
        logits = _dot(r, l3w_ref[...]) + l3b_ref[...] + le_ref[0, 0]
        m = jnp.max(logits, axis=1, keepdims=True)
        lse = m + jnp.log(jnp.sum(jnp.exp(logits - m), axis=1, keepdims=True))
        out_ref[...] = logits - lse


def _make_prep():
    return pl.pallas_call(
        _prep_body,
        grid=(2, NB),
        in_specs=[
            pl.BlockSpec((BLK, D), lambda c, rb: (rb, 0)),
            pl.BlockSpec((1, D, H), lambda c, rb: (c, 0, 0)),
            pl.BlockSpec((1, D, D), lambda c, rb: (c, 0, 0)),
            pl.BlockSpec((1, 1, H), lambda c, rb: (c, 0, 0)),
        ],
        out_specs=[
            pl.BlockSpec((1, BLK, H), lambda c, rb: (c, rb, 0)),
            pl.BlockSpec((1, BLK, D), lambda c, rb: (c, rb, 0)),
        ],
        out_shape=[
            jax.ShapeDtypeStruct((2, NPAD, H), _F32),
            jax.ShapeDtypeStruct((2, NPAD, D), _F32),
        ],
    )


def _make_comb(do_readout):
    in_specs = [
        pl.BlockSpec((2, BLK, H), lambda c, rb: (0, rb, 0)),
        pl.BlockSpec((2, BLK, D), lambda c, rb: (0, rb, 0)),
        pl.BlockSpec((1, D, H), lambda c, rb: (c, 0, 0)),
        pl.BlockSpec((1, D, D), lambda c, rb: (c, 0, 0)),
        pl.BlockSpec((1, 1, H), lambda c, rb: (c, 0, 0)),
    ]
    out_specs = [
        pl.BlockSpec((1, BLK, H), lambda c, rb: (c, rb, 0)),
        pl.BlockSpec((1, BLK, D), lambda c, rb: (c, rb, 0)),
    ]
    out_shape = [
        jax.ShapeDtypeStruct((2, NPAD, H), _F32),
        jax.ShapeDtypeStruct((2, NPAD, D), _F32),
    ]
    scratch = []
    if do_readout:
        in_specs.append(pl.BlockSpec((1, BLK, 1), lambda c, rb: (rb, 0, 0)))
        out_specs.append(pl.BlockSpec((G, 2 * D), lambda c, rb: (0, 0)))
        out_shape.append(jax.ShapeDtypeStruct((G, 2 * D), _F32))
        scratch = [pltpu.VMEM((G, D), _F32)] * 3
    return pl.pallas_call(
        functools.partial(_comb_body, do_readout),
        grid=(2, NB),
        in_specs=in_specs,
        out_specs=out_specs,
        out_shape=out_shape,
        scratch_shapes=scratch,
    )


def _make_final():
    return pl.pallas_call(
        _final_body,
        grid=(NB,),
        in_specs=[
            pl.BlockSpec((2, BLK, H), lambda rb: (0, rb, 0)),
            pl.BlockSpec((2, BLK, D), lambda rb: (0, rb, 0)),
            pl.BlockSpec((1, BLK, 1), lambda rb: (rb, 0, 0)),
            pl.BlockSpec((G, 2 * D), lambda rb: (0, 0)),
            pl.BlockSpec((2 * D, D), lambda rb: (0, 0)),
            pl.BlockSpec((1, D), lambda rb: (0, 0)),
            pl.BlockSpec((D, H), lambda rb: (0, 0)),
            pl.BlockSpec((1, H), lambda rb: (0, 0)),
            pl.BlockSpec((H, CL), lambda rb: (0, 0)),
            pl.BlockSpec((1, CL), lambda rb: (0, 0)),
            pl.BlockSpec((1, 1), lambda rb: (0, 0)),
        ],
        out_specs=pl.BlockSpec((G, CL), lambda rb: (0, 0)),
        out_shape=jax.ShapeDtypeStruct((G, CL), _F32),
        scratch_shapes=[pltpu.VMEM((G, D), _F32)] * 3,
    )


def kernel(x, edge_index, batch, hom_mask, het_mask,
           hom_W, hom_b, het_W, het_b,
           lin1_W, lin1_b, lin2_W, lin2_b, lin3_W, lin3_b, last_epoch):
    src = edge_index[0].astype(jnp.int32)
    dst = edge_index[1].astype(jnp.int32)
    pad_idx = (jnp.arange(EPAD - E, dtype=jnp.int32) % N)
    src2 = jnp.concatenate([src, pad_idx])
    dst2 = jnp.concatenate([dst, pad_idx])
    epk = (src2 | (dst2 << 16)).reshape(ECH, K)
    zpad_e = jnp.zeros((EPAD - E,), _F32)
    masks = jnp.stack([
        jnp.concatenate([hom_mask.astype(_F32), zpad_e]),
        jnp.concatenate([het_mask.astype(_F32), zpad_e]),
    ]).reshape(2, ECH, K)
    xp = jnp.zeros((NPAD, D), _F32).at[:N].set(x.astype(_F32))
    batchp = jnp.concatenate(
        [batch.astype(jnp.int32), jnp.full((NPAD - N,), G, jnp.int32)]
    ).reshape(NB, BLK, 1)

    def lw(l):
        w0 = jnp.stack([hom_W[l, 0], het_W[l, 0]]).astype(_F32)
        w12 = jnp.concatenate([
            jnp.stack([hom_W[l, 1], het_W[l, 1]]),
            jnp.stack([hom_W[l, 2], het_W[l, 2]]),
        ], axis=-1).astype(_F32)
        b0 = jnp.stack([hom_b[l], het_b[l]]).astype(_F32).reshape(2, 1, H)
        return w0, w12, b0

    deg_k = _make_deg()
    layer_k = _make_layer()
    prep_k = _make_prep()
    comb_k = _make_comb(False)
    combr_k = _make_comb(True)
    final_k = _make_final()

    invd = deg_k(epk, masks)

    w0, w12, b0 = lw(0)
    z, ba = prep_k(xp, w0, w12, b0)
    masks2 = masks.reshape(2 * ECH, K)
    invd2 = invd
    s_agg, _ = layer_k(ba.reshape(2 * NPAD, D), epk, masks2, invd2)

    w0, w12, b0 = lw(1)
    z, ba = comb_k(z, s_agg, w0, w12, b0)
    s_agg, _ = layer_k(ba.reshape(2 * NPAD, D), epk, masks2, invd2)

    w0, w12, b0 = lw(2)
    z, ba, ro1 = combr_k(z, s_agg, w0, w12, b0, batchp)
    s_agg, _ = layer_k(ba.reshape(2 * NPAD, D), epk, masks2, invd2)

    return final_k(z, s_agg, batchp, ro1,
                   lin1_W.astype(_F32), lin1_b.astype(_F32).reshape(1, D),
                   lin2_W.astype(_F32), lin2_b.astype(_F32).reshape(1, H),
                   lin3_W.astype(_F32), lin3_b.astype(_F32).reshape(1, CL),
                   jnp.asarray(last_epoch, _F32).reshape(1, 1))
